# baseline (speedup 1.0000x reference)
import functools

import jax
import jax.numpy as jnp
from jax import lax
from jax.experimental import pallas as pl
from jax.experimental.pallas import tpu as pltpu
from jax.experimental.pallas import tpu_sc as plsc

_D = 32
_B = 4096
_T = 200
_NC, _NS = 2, 16
_NW = _NC * _NS
_UNITS = (_B // 128) * _T // 4
_UPW = _UNITS // _NW
_SCALE = float(_D) ** 0.5
_GBYTES = 512 * _D * 4
_WBYTES = 4 * 4 * 8 * 128 * 4


def _compute(idxb, rowsb, ob):
    iota16 = lax.iota(jnp.int32, 16)

    def st_body(st, _):
        def lg_body(lg, _):
            iv = idxb[st, pl.ds(lg * 16, 16)]
            sc = jnp.where(iv == 0, 0.0, _SCALE).astype(jnp.float32)
            rvec = st * 128 + lg * 16 + iota16
            for d in range(_D):
                v = plsc.load_gather(rowsb, [rvec, jnp.full((16,), d, jnp.int32)])
                ob[st, d // 8, d % 8, pl.ds(lg * 16, 16)] = v * sc
            return 0

        return lax.fori_loop(0, 8, lg_body, 0)

    lax.fori_loop(0, 4, st_body, 0)


def _body(table, xv, out5, idx0, idx1, rows0, rows1, ob0, ob1,
          gsem0, gsem1, wsem0, wsem1):
    c = lax.axis_index("c")
    s = lax.axis_index("s")
    base = (s * _NC + c) * _UPW

    def decode(g):
        return g >> 6, (g >> 1) & 31, g & 1

    def start(g, idxb, rowsb, gsem):
        tb, bb, h = decode(g)
        pltpu.sync_copy(xv.at[tb, bb, pl.ds(h * 4, 4), :], idxb)
        for k in range(4):
            pltpu.async_copy(
                table.at[idxb.at[k]], rowsb.at[pl.ds(k * 128, 128), :], gsem
            )

    def drain(sem, rowsb):
        pltpu.make_async_copy(table.at[pl.ds(0, 512), :], rowsb, sem).wait()

    def finish(g, idxb, rowsb, ob, gsem, wsem, wait_w):
        drain(gsem, rowsb)

        @pl.when(wait_w)
        def _():
            drain(wsem, rowsb)

        _compute(idxb, rowsb, ob)
        tb, bb, h = decode(g)
        t0 = tb * 8 + h * 4
        for st in range(4):
            for db in range(4):
                pltpu.async_copy(ob.at[st, db], out5.at[t0 + st, db, bb], wsem)

    start(base, idx0, rows0, gsem0)

    def pair(i, _):
        g0 = base + i * 2
        start(g0 + 1, idx1, rows1, gsem1)
        finish(g0, idx0, rows0, ob0, gsem0, wsem0, i > 0)

        @pl.when(i < _UPW // 2 - 1)
        def _():
            start(g0 + 2, idx0, rows0, gsem0)

        finish(g0 + 1, idx1, rows1, ob1, gsem1, wsem1, i > 0)
        return 0

    lax.fori_loop(0, _UPW // 2, pair, 0)
    drain(wsem0, rows0)
    drain(wsem1, rows1)


@functools.cache
def _sc_call():
    return functools.partial(
        pl.kernel,
        out_type=jax.ShapeDtypeStruct((_T, 4, _B // 128, 8, 128), jnp.float32),
        compiler_params=pltpu.CompilerParams(
            needs_layout_passes=False, use_tc_tiling_on_sc=False
        ),
        mesh=plsc.VectorSubcoreMesh(
            core_axis_name="c", subcore_axis_name="s", num_cores=_NC, num_subcores=_NS
        ),
        scratch_types=[
            pltpu.VMEM((4, 128), jnp.int32),
            pltpu.VMEM((4, 128), jnp.int32),
            pltpu.VMEM((512, _D), jnp.float32),
            pltpu.VMEM((512, _D), jnp.float32),
            pltpu.VMEM((4, 4, 8, 128), jnp.float32),
            pltpu.VMEM((4, 4, 8, 128), jnp.float32),
            pltpu.SemaphoreType.DMA,
            pltpu.SemaphoreType.DMA,
            pltpu.SemaphoreType.DMA,
            pltpu.SemaphoreType.DMA,
        ],
    )(_body)


def kernel(shared_weights, x):
    xv = x.T.reshape(_T // 8, 8, _B // 128, 128).transpose(0, 2, 1, 3)
    out5 = _sc_call()(shared_weights, xv)
    return out5.transpose(2, 4, 0, 1, 3).reshape(_B, _T, _D)


# speedup vs baseline: 1.3643x; 1.3643x over previous
import functools

import jax
import jax.numpy as jnp
from jax import lax
from jax.experimental import pallas as pl
from jax.experimental.pallas import tpu as pltpu
from jax.experimental.pallas import tpu_sc as plsc

_D = 32
_B = 4096
_T = 200
_NC, _NS = 2, 16
_NW = _NC * _NS
_UNITS = (_B // 128) * _T // 4
_UPW = _UNITS // _NW
_SCALE = float(_D) ** 0.5
_GBYTES = 512 * _D * 4
_WBYTES = 4 * 4 * 8 * 128 * 4


def _compute(idxb, rowsb, ob):
    iota16 = lax.iota(jnp.int32, 16)
    dvecs = [jnp.full((16,), d, jnp.int32) for d in range(_D)]

    def st_body(st, _):
        base = st * 128 + iota16
        for lg in range(8):
            iv = idxb[st, pl.ds(lg * 16, 16)]
            sc = jnp.where(iv == 0, 0.0, _SCALE).astype(jnp.float32)
            rvec = base + lg * 16
            vs = [plsc.load_gather(rowsb, [rvec, dvecs[d]]) for d in range(_D)]
            for d in range(_D):
                ob[st, d // 8, d % 8, pl.ds(lg * 16, 16)] = vs[d] * sc
        return 0

    lax.fori_loop(0, 4, st_body, 0)


def _body(table, xv, out5, idx0, idx1, rows0, rows1, ob0, ob1,
          gsem0, gsem1, wsem0, wsem1):
    c = lax.axis_index("c")
    s = lax.axis_index("s")
    base = (s * _NC + c) * _UPW

    def decode(g):
        return g >> 6, (g >> 1) & 31, g & 1

    def start(g, idxb, rowsb, gsem):
        tb, bb, h = decode(g)
        pltpu.sync_copy(xv.at[tb, bb, pl.ds(h * 4, 4), :], idxb)
        for k in range(4):
            pltpu.async_copy(
                table.at[idxb.at[k]], rowsb.at[pl.ds(k * 128, 128), :], gsem
            )

    def drain(sem, rowsb):
        pltpu.make_async_copy(table.at[pl.ds(0, 512), :], rowsb, sem).wait()

    def finish(g, idxb, rowsb, ob, gsem, wsem, wait_w):
        drain(gsem, rowsb)

        @pl.when(wait_w)
        def _():
            drain(wsem, rowsb)

        _compute(idxb, rowsb, ob)
        tb, bb, h = decode(g)
        t0 = tb * 8 + h * 4
        for st in range(4):
            for db in range(4):
                pltpu.async_copy(ob.at[st, db], out5.at[t0 + st, db, bb], wsem)

    start(base, idx0, rows0, gsem0)

    def pair(i, _):
        g0 = base + i * 2
        start(g0 + 1, idx1, rows1, gsem1)
        finish(g0, idx0, rows0, ob0, gsem0, wsem0, i > 0)

        @pl.when(i < _UPW // 2 - 1)
        def _():
            start(g0 + 2, idx0, rows0, gsem0)

        finish(g0 + 1, idx1, rows1, ob1, gsem1, wsem1, i > 0)
        return 0

    lax.fori_loop(0, _UPW // 2, pair, 0)
    drain(wsem0, rows0)
    drain(wsem1, rows1)


@functools.cache
def _sc_call():
    return functools.partial(
        pl.kernel,
        out_type=jax.ShapeDtypeStruct((_T, 4, _B // 128, 8, 128), jnp.float32),
        compiler_params=pltpu.CompilerParams(
            needs_layout_passes=False, use_tc_tiling_on_sc=False
        ),
        mesh=plsc.VectorSubcoreMesh(
            core_axis_name="c", subcore_axis_name="s", num_cores=_NC, num_subcores=_NS
        ),
        scratch_types=[
            pltpu.VMEM((4, 128), jnp.int32),
            pltpu.VMEM((4, 128), jnp.int32),
            pltpu.VMEM((512, _D), jnp.float32),
            pltpu.VMEM((512, _D), jnp.float32),
            pltpu.VMEM((4, 4, 8, 128), jnp.float32),
            pltpu.VMEM((4, 4, 8, 128), jnp.float32),
            pltpu.SemaphoreType.DMA,
            pltpu.SemaphoreType.DMA,
            pltpu.SemaphoreType.DMA,
            pltpu.SemaphoreType.DMA,
        ],
    )(_body)


def kernel(shared_weights, x):
    xv = x.T.reshape(_T // 8, 8, _B // 128, 128).transpose(0, 2, 1, 3)
    out5 = _sc_call()(shared_weights, xv)
    return out5.transpose(2, 4, 0, 1, 3).reshape(_B, _T, _D)


# speedup vs baseline: 1.5431x; 1.1311x over previous
import functools

import jax
import jax.numpy as jnp
from jax import lax
from jax.experimental import pallas as pl
from jax.experimental.pallas import tpu as pltpu
from jax.experimental.pallas import tpu_sc as plsc

_D = 32
_B = 4096
_T = 200
_NC, _NS = 2, 16
_NW = _NC * _NS
_UNITS = (_B // 128) * _T // 4
_UPW = _UNITS // _NW
_SCALE = float(_D) ** 0.5
_GBYTES = 512 * _D * 4
_WBYTES = 4 * 4 * 8 * 128 * 4


def _compute(idxb, rowsb, ob):
    iota16 = lax.iota(jnp.int32, 16)
    dvecs = [jnp.full((16,), d, jnp.int32) for d in range(_D)]

    def st_body(st, _):
        base = st * 128 + iota16
        for lg in range(8):
            iv = idxb[st, pl.ds(lg * 16, 16)]
            sc = jnp.where(iv == 0, 0.0, _SCALE).astype(jnp.float32)
            rvec = base + lg * 16
            vs = [plsc.load_gather(rowsb, [rvec, dvecs[d]]) for d in range(_D)]
            for d in range(_D):
                ob[st, d // 8, d % 8, pl.ds(lg * 16, 16)] = vs[d] * sc
        return 0

    lax.fori_loop(0, 4, st_body, 0)


def _body(table, xv, out5, idx0, idx1, rows0, rows1, ob0, ob1,
          gsem0, gsem1, wsem0, wsem1):
    c = lax.axis_index("c")
    s = lax.axis_index("s")
    base = (s * _NC + c) * _UPW

    def decode(g):
        return g >> 6, (g >> 1) & 31, g & 1

    def start(g, idxb, rowsb, gsem):
        tb, bb, h = decode(g)
        pltpu.sync_copy(xv.at[tb, bb, pl.ds(h * 4, 4), :], idxb)
        for k in range(4):
            pltpu.async_copy(
                table.at[idxb.at[k]], rowsb.at[pl.ds(k * 128, 128), :], gsem
            )

    def drain(sem, rowsb):
        pltpu.make_async_copy(table.at[pl.ds(0, 512), :], rowsb, sem).wait()

    def finish(g, idxb, rowsb, ob, gsem, wsem, wait_w):
        drain(gsem, rowsb)

        @pl.when(wait_w)
        def _():
            drain(wsem, rowsb)

        _compute(idxb, rowsb, ob)
        tb, bb, h = decode(g)
        t0 = tb * 8 + h * 4
        for st in range(4):
            for db in range(4):
                pltpu.async_copy(ob.at[st, db], out5.at[t0 + st, db, bb], wsem)

    start(base, idx0, rows0, gsem0)

    def pair(i, _):
        g0 = base + i * 2
        start(g0 + 1, idx1, rows1, gsem1)
        finish(g0, idx0, rows0, ob0, gsem0, wsem0, i > 0)

        @pl.when(i < _UPW // 2 - 1)
        def _():
            start(g0 + 2, idx0, rows0, gsem0)

        finish(g0 + 1, idx1, rows1, ob1, gsem1, wsem1, i > 0)
        return 0

    lax.fori_loop(0, _UPW // 2, pair, 0)
    drain(wsem0, rows0)
    drain(wsem1, rows1)


@functools.cache
def _sc_call():
    return functools.partial(
        pl.kernel,
        out_type=jax.ShapeDtypeStruct((_T, 4, _B // 128, 8, 128), jnp.float32),
        compiler_params=pltpu.CompilerParams(
            needs_layout_passes=False, use_tc_tiling_on_sc=False
        ),
        mesh=plsc.VectorSubcoreMesh(
            core_axis_name="c", subcore_axis_name="s", num_cores=_NC, num_subcores=_NS
        ),
        scratch_types=[
            pltpu.VMEM((4, 128), jnp.int32),
            pltpu.VMEM((4, 128), jnp.int32),
            pltpu.VMEM((512, _D), jnp.float32),
            pltpu.VMEM((512, _D), jnp.float32),
            pltpu.VMEM((4, 4, 8, 128), jnp.float32),
            pltpu.VMEM((4, 4, 8, 128), jnp.float32),
            pltpu.SemaphoreType.DMA,
            pltpu.SemaphoreType.DMA,
            pltpu.SemaphoreType.DMA,
            pltpu.SemaphoreType.DMA,
        ],
    )(_body)


_TC_BV = 8192


def _tc_transpose_body(tin, tout):
    t = tin[...].T.reshape(_TC_BV // 4, 4, _D)
    tout[...] = jnp.concatenate([t[:, j, :] for j in range(4)], axis=1)


@functools.cache
def _tc_transpose_call():
    grid = -(-1000000 // _TC_BV)
    return pl.pallas_call(
        _tc_transpose_body,
        grid=(grid,),
        in_specs=[pl.BlockSpec((32, _TC_BV), lambda i: (0, i))],
        out_specs=pl.BlockSpec((_TC_BV // 4, 128), lambda i: (i, 0)),
        out_shape=jax.ShapeDtypeStruct((250000, 128), jnp.float32),
    )


def kernel(shared_weights, x):
    xv = x.T.reshape(_T // 8, 8, _B // 128, 128).transpose(0, 2, 1, 3)
    table = _tc_transpose_call()(shared_weights.T).reshape(1000000, _D)
    out5 = _sc_call()(table, xv)
    return out5.transpose(2, 4, 0, 1, 3).reshape(_B, _T, _D)


# speedup vs baseline: 1.8982x; 1.2301x over previous
import functools

import jax
import jax.numpy as jnp
from jax import lax
from jax.experimental import pallas as pl
from jax.experimental.pallas import tpu as pltpu
from jax.experimental.pallas import tpu_sc as plsc

_D = 32
_B = 4096
_T = 200
_NC, _NS = 2, 16
_NW = _NC * _NS
_UNITS = (_B // 128) * _T // 4
_UPW = _UNITS // _NW
_SCALE = float(_D) ** 0.5
_GBYTES = 512 * _D * 4
_WBYTES = 4 * 4 * 8 * 128 * 4


def _compute(idxb, rowsb, ob):
    iota16 = lax.iota(jnp.int32, 16)
    dvecs = [jnp.full((16,), d, jnp.int32) for d in range(_D)]

    def st_body(st, _):
        base = st * 128 + iota16
        for lg in range(8):
            iv = idxb[st, pl.ds(lg * 16, 16)]
            sc = jnp.where(iv == 0, 0.0, _SCALE).astype(jnp.float32)
            rvec = base + lg * 16
            vs = [plsc.load_gather(rowsb, [rvec, dvecs[d]]) for d in range(_D)]
            for d in range(_D):
                ob[st, d // 8, d % 8, pl.ds(lg * 16, 16)] = vs[d] * sc
        return 0

    lax.fori_loop(0, 4, st_body, 0)


def _body(table, xv, out5, idx0, idx1, rows0, rows1, ob0, ob1,
          gsem0, gsem1, wsem0, wsem1):
    c = lax.axis_index("c")
    s = lax.axis_index("s")
    base = (s * _NC + c) * _UPW

    def decode(g):
        return g >> 6, (g >> 1) & 31, g & 1

    def start(g, idxb, rowsb, gsem):
        tb, bb, h = decode(g)
        pltpu.sync_copy(xv.at[tb, bb, pl.ds(h * 4, 4), :], idxb)
        for k in range(4):
            for lg in range(8):
                sl = pl.ds(lg * 16, 16)
                idxb[k, sl] = idxb[k, sl] << 2
        for k in range(4):
            pltpu.async_copy(
                table.at[idxb.at[k]], rowsb.at[pl.ds(k * 128, 128), :], gsem
            )

    def drain(sem, rowsb):
        pltpu.make_async_copy(table.at[pl.ds(0, 512), :], rowsb, sem).wait()

    def finish(g, idxb, rowsb, ob, gsem, wsem, wait_w):
        drain(gsem, rowsb)

        @pl.when(wait_w)
        def _():
            drain(wsem, rowsb)

        _compute(idxb, rowsb, ob)
        tb, bb, h = decode(g)
        t0 = tb * 8 + h * 4
        for st in range(4):
            for db in range(4):
                pltpu.async_copy(ob.at[st, db], out5.at[t0 + st, db, bb], wsem)

    start(base, idx0, rows0, gsem0)

    def pair(i, _):
        g0 = base + i * 2
        start(g0 + 1, idx1, rows1, gsem1)
        finish(g0, idx0, rows0, ob0, gsem0, wsem0, i > 0)

        @pl.when(i < _UPW // 2 - 1)
        def _():
            start(g0 + 2, idx0, rows0, gsem0)

        finish(g0 + 1, idx1, rows1, ob1, gsem1, wsem1, i > 0)
        return 0

    lax.fori_loop(0, _UPW // 2, pair, 0)
    drain(wsem0, rows0)
    drain(wsem1, rows1)


@functools.cache
def _sc_call():
    return functools.partial(
        pl.kernel,
        out_type=jax.ShapeDtypeStruct((_T, 4, _B // 128, 8, 128), jnp.float32),
        compiler_params=pltpu.CompilerParams(
            needs_layout_passes=False, use_tc_tiling_on_sc=False
        ),
        mesh=plsc.VectorSubcoreMesh(
            core_axis_name="c", subcore_axis_name="s", num_cores=_NC, num_subcores=_NS
        ),
        scratch_types=[
            pltpu.VMEM((4, 128), jnp.int32),
            pltpu.VMEM((4, 128), jnp.int32),
            pltpu.VMEM((512, _D), jnp.float32),
            pltpu.VMEM((512, _D), jnp.float32),
            pltpu.VMEM((4, 4, 8, 128), jnp.float32),
            pltpu.VMEM((4, 4, 8, 128), jnp.float32),
            pltpu.SemaphoreType.DMA,
            pltpu.SemaphoreType.DMA,
            pltpu.SemaphoreType.DMA,
            pltpu.SemaphoreType.DMA,
        ],
    )(_body)


_TC_BV = 8192


def _tc_transpose_body(tin, tout):
    tout[:, :, 0:_D] = tin[...].T.reshape(_TC_BV // 8, 8, _D)


@functools.cache
def _tc_transpose_call():
    grid = -(-1000000 // _TC_BV)
    return pl.pallas_call(
        _tc_transpose_body,
        grid=(grid,),
        in_specs=[pl.BlockSpec((32, _TC_BV), lambda i: (0, i))],
        out_specs=pl.BlockSpec((_TC_BV // 8, 8, 128), lambda i: (i, 0, 0)),
        out_shape=jax.ShapeDtypeStruct((125000, 8, 128), jnp.float32),
    )


def kernel(shared_weights, x):
    xv = x.T.reshape(_T // 8, 8, _B // 128, 128).transpose(0, 2, 1, 3)
    table = _tc_transpose_call()(shared_weights.T).reshape(4000000, _D)
    out5 = _sc_call()(table, xv)
    return out5.transpose(2, 4, 0, 1, 3).reshape(_B, _T, _D)


# speedup vs baseline: 1.9047x; 1.0034x over previous
import functools

import jax
import jax.numpy as jnp
from jax import lax
from jax.experimental import pallas as pl
from jax.experimental.pallas import tpu as pltpu
from jax.experimental.pallas import tpu_sc as plsc

_D = 32
_B = 4096
_T = 200
_NC, _NS = 2, 16
_NW = _NC * _NS
_UNITS = (_B // 128) * _T // 4
_UPW = _UNITS // _NW
_SCALE = float(_D) ** 0.5
_GBYTES = 512 * _D * 4
_WBYTES = 4 * 4 * 8 * 128 * 4


def _compute(idxb, rowsb, ob):
    iota16 = lax.iota(jnp.int32, 16)
    dvecs = [jnp.full((16,), d, jnp.int32) for d in range(_D)]

    def st_body(st, _):
        base = st * 128 + iota16
        for lg in range(8):
            iv = idxb[st, pl.ds(lg * 16, 16)]
            sc = jnp.where(iv == 0, 0.0, _SCALE).astype(jnp.float32)
            rvec = base + lg * 16
            vs = [plsc.load_gather(rowsb, [rvec, dvecs[d]]) for d in range(_D)]
            for d in range(_D):
                ob[st, d // 8, d % 8, pl.ds(lg * 16, 16)] = vs[d] * sc
        return 0

    lax.fori_loop(0, 4, st_body, 0)


def _body(table, xv, out5, idx0, idx1, rows0, rows1, ob0, ob1,
          gsem0, gsem1, wsem0, wsem1):
    c = lax.axis_index("c")
    s = lax.axis_index("s")
    base = (s * _NC + c) * _UPW

    def decode(g):
        return g >> 6, (g >> 1) & 31, g & 1

    def start(g, idxb, rowsb, gsem):
        tb, bb, h = decode(g)
        pltpu.sync_copy(xv.at[tb, bb, pl.ds(h * 4, 4), :], idxb)
        for k in range(4):
            for lg in range(8):
                sl = pl.ds(lg * 16, 16)
                idxb[k, sl] = idxb[k, sl] << 2
        for k in range(4):
            pltpu.async_copy(
                table.at[idxb.at[k]], rowsb.at[pl.ds(k * 128, 128), :], gsem
            )

    def drain(sem, rowsb):
        pltpu.make_async_copy(table.at[pl.ds(0, 512), :], rowsb, sem).wait()

    def finish(g, idxb, rowsb, ob, gsem, wsem, wait_w):
        drain(gsem, rowsb)

        @pl.when(wait_w)
        def _():
            drain(wsem, rowsb)

        _compute(idxb, rowsb, ob)
        tb, bb, h = decode(g)
        t0 = tb * 8 + h * 4
        pltpu.async_copy(ob, out5.at[pl.ds(t0, 4), :, bb], wsem)

    start(base, idx0, rows0, gsem0)

    def pair(i, _):
        g0 = base + i * 2
        start(g0 + 1, idx1, rows1, gsem1)
        finish(g0, idx0, rows0, ob0, gsem0, wsem0, i > 0)

        @pl.when(i < _UPW // 2 - 1)
        def _():
            start(g0 + 2, idx0, rows0, gsem0)

        finish(g0 + 1, idx1, rows1, ob1, gsem1, wsem1, i > 0)
        return 0

    lax.fori_loop(0, _UPW // 2, pair, 0)
    drain(wsem0, rows0)
    drain(wsem1, rows1)


@functools.cache
def _sc_call():
    return functools.partial(
        pl.kernel,
        out_type=jax.ShapeDtypeStruct((_T, 4, _B // 128, 8, 128), jnp.float32),
        compiler_params=pltpu.CompilerParams(
            needs_layout_passes=False, use_tc_tiling_on_sc=False
        ),
        mesh=plsc.VectorSubcoreMesh(
            core_axis_name="c", subcore_axis_name="s", num_cores=_NC, num_subcores=_NS
        ),
        scratch_types=[
            pltpu.VMEM((4, 128), jnp.int32),
            pltpu.VMEM((4, 128), jnp.int32),
            pltpu.VMEM((512, _D), jnp.float32),
            pltpu.VMEM((512, _D), jnp.float32),
            pltpu.VMEM((4, 4, 8, 128), jnp.float32),
            pltpu.VMEM((4, 4, 8, 128), jnp.float32),
            pltpu.SemaphoreType.DMA,
            pltpu.SemaphoreType.DMA,
            pltpu.SemaphoreType.DMA,
            pltpu.SemaphoreType.DMA,
        ],
    )(_body)


_TC_BV = 8192


def _tc_transpose_body(tin, tout):
    tout[:, :, 0:_D] = tin[...].T.reshape(_TC_BV // 8, 8, _D)


@functools.cache
def _tc_transpose_call():
    grid = -(-1000000 // _TC_BV)
    return pl.pallas_call(
        _tc_transpose_body,
        grid=(grid,),
        in_specs=[pl.BlockSpec((32, _TC_BV), lambda i: (0, i))],
        out_specs=pl.BlockSpec((_TC_BV // 8, 8, 128), lambda i: (i, 0, 0)),
        out_shape=jax.ShapeDtypeStruct((125000, 8, 128), jnp.float32),
    )


def kernel(shared_weights, x):
    xv = x.T.reshape(_T // 8, 8, _B // 128, 128).transpose(0, 2, 1, 3)
    table = _tc_transpose_call()(shared_weights.T).reshape(4000000, _D)
    out5 = _sc_call()(table, xv)
    return out5.transpose(2, 4, 0, 1, 3).reshape(_B, _T, _D)
